# baseline (speedup 1.0000x reference)
import functools

import jax
import jax.numpy as jnp
from jax import lax
from jax.experimental import pallas as pl
from jax.experimental.pallas import tpu as pltpu
from jax.experimental.pallas import tpu_sc as plsc

N_NODES = 10000
N_PAD = 10240
DIM = 64
CHUNK = 128
G = 5
NC = 2
NS = 16
NW = NC * NS


PDIM = DIM // 2


def _lrelu(v):
    return jnp.where(v >= 0, v, 0.01 * v)


def _pack_cols(x):
    rnd = jnp.uint32(0x8000)
    lo = (lax.bitcast_convert_type(x[:, :PDIM], jnp.uint32) + rnd) >> 16
    hi = ((lax.bitcast_convert_type(x[:, PDIM:], jnp.uint32) + rnd)
          & jnp.uint32(0xFFFF0000))
    return lax.bitcast_convert_type(lo | hi, jnp.float32)


def _unpack_cols(pk):
    u = lax.bitcast_convert_type(pk, jnp.uint32)
    lo = lax.bitcast_convert_type(u << 16, jnp.float32)
    hi = lax.bitcast_convert_type(u & jnp.uint32(0xFFFF0000), jnp.float32)
    return lo, hi


def _tc_stats(x):
    r, d = x.shape
    blk = 8000 if r % 8000 == 0 else (2000 if r % 2000 == 0 else 1000)
    grid = r // blk

    def kern(x_ref, s_ref, q_ref):
        @pl.when(pl.program_id(0) == 0)
        def _():
            s_ref[...] = jnp.zeros_like(s_ref)
            q_ref[...] = jnp.zeros_like(q_ref)

        xb = x_ref[...]
        s = jnp.sum(xb, axis=0, keepdims=True)
        q = jnp.sum(xb * xb, axis=0, keepdims=True)
        s_ref[...] += jnp.broadcast_to(s, (8, d))
        q_ref[...] += jnp.broadcast_to(q, (8, d))

    return pl.pallas_call(
        kern,
        grid=(grid,),
        in_specs=[pl.BlockSpec((blk, d), lambda i: (i, 0))],
        out_specs=(pl.BlockSpec((8, d), lambda i: (0, 0)),
                   pl.BlockSpec((8, d), lambda i: (0, 0))),
        out_shape=(jax.ShapeDtypeStruct((8, d), jnp.float32),
                   jax.ShapeDtypeStruct((8, d), jnp.float32)),
    )(x)


def _tc_bn_mm(x, s, q, g, b, w, c):
    r, din = x.shape
    dout = w.shape[1]
    blk = 8000 if r % 8000 == 0 else (2000 if r % 2000 == 0 else 1000)
    grid = r // blk
    n = float(r)

    def kern(x_ref, s_ref, q_ref, g_ref, b_ref, w_ref, c_ref,
             o_ref, so_ref, qo_ref):
        @pl.when(pl.program_id(0) == 0)
        def _():
            so_ref[...] = jnp.zeros_like(so_ref)
            qo_ref[...] = jnp.zeros_like(qo_ref)

        mu = s_ref[...][0:1, :] / n
        var = q_ref[...][0:1, :] / n - mu * mu
        inv = lax.rsqrt(var + 1e-5)
        h = (x_ref[...] - mu) * (inv * g_ref[...]) + b_ref[...]
        y = jnp.dot(h.astype(jnp.bfloat16), w_ref[...].astype(jnp.bfloat16),
                    preferred_element_type=jnp.float32)
        o = _lrelu(y + c_ref[...])
        o_ref[...] = o
        so_ref[...] += jnp.broadcast_to(jnp.sum(o, axis=0, keepdims=True),
                                        (8, dout))
        qo_ref[...] += jnp.broadcast_to(jnp.sum(o * o, axis=0, keepdims=True),
                                        (8, dout))

    full = lambda shape: pl.BlockSpec(shape, lambda i: tuple(0 for _ in shape))
    return pl.pallas_call(
        kern,
        grid=(grid,),
        in_specs=[pl.BlockSpec((blk, din), lambda i: (i, 0)),
                  full((8, din)), full((8, din)),
                  full((1, din)), full((1, din)),
                  full((din, dout)), full((1, dout))],
        out_specs=(pl.BlockSpec((blk, dout), lambda i: (i, 0)),
                   full((8, dout)), full((8, dout))),
        out_shape=(jax.ShapeDtypeStruct((r, dout), jnp.float32),
                   jax.ShapeDtypeStruct((8, dout), jnp.float32),
                   jax.ShapeDtypeStruct((8, dout), jnp.float32)),
    )(x, s, q, g.reshape(1, din), b.reshape(1, din), w, c.reshape(1, dout))


def _mlp(p, x):
    s, q = _tc_stats(x)
    h, s2, q2 = _tc_bn_mm(x, s, q, p['g1'], p['b1'], p['W1'], p['c1'])
    return _tc_bn_mm(h, s2, q2, p['g2'], p['b2'], p['W2'], p['c2'])[0]


def _tc_edge(xs, xd, ea, wa, wb, wc, be):
    e = xs.shape[0]
    blk = 8000
    grid = e // blk

    def kern(xs_ref, xd_ref, ea_ref, wa_ref, wb_ref, wc_ref, be_ref,
             en_ref, eo_ref):
        xsl, xsh = _unpack_cols(xs_ref[...])
        xdl, xdh = _unpack_cols(xd_ref[...])
        wa = wa_ref[...]
        wb = wb_ref[...]
        y = (jnp.dot(xsl, wa[:PDIM], preferred_element_type=jnp.float32)
             + jnp.dot(xsh, wa[PDIM:], preferred_element_type=jnp.float32)
             + jnp.dot(xdl, wb[:PDIM], preferred_element_type=jnp.float32)
             + jnp.dot(xdh, wb[PDIM:], preferred_element_type=jnp.float32)
             + jnp.dot(ea_ref[...], wc_ref[...], preferred_element_type=jnp.float32)
             + be_ref[...])
        en = _lrelu(y)
        en_ref[...] = en
        eo_ref[...] = ea_ref[...] + en

    row = lambda i: (i, 0)
    zero = lambda i: (0, 0)
    return pl.pallas_call(
        kern,
        grid=(grid,),
        in_specs=[pl.BlockSpec((blk, PDIM), row)] * 2 +
                 [pl.BlockSpec((blk, DIM), row)] +
                 [pl.BlockSpec((DIM, DIM), zero)] * 3 +
                 [pl.BlockSpec((1, DIM), zero)],
        out_specs=(pl.BlockSpec((blk, DIM), row), pl.BlockSpec((blk, DIM), row)),
        out_shape=(jax.ShapeDtypeStruct((e, DIM), jnp.float32),
                   jax.ShapeDtypeStruct((e, DIM), jnp.float32)),
    )(xs, xd, ea, wa, wb, wc, be.reshape(1, DIM))


def _tc_node(x, parts, cnts, wn1, wn2, bn):
    blk = 2000
    grid = N_NODES // blk

    def kern(x_ref, p_ref, c_ref, w1_ref, w2_ref, bn_ref, o_ref, pk_ref):
        psum = p_ref[0] + p_ref[1]
        cnt = (c_ref[0] + c_ref[1])[:, 0:1]
        agg = psum / jnp.maximum(cnt, 1.0)
        y = (jnp.dot(x_ref[...], w1_ref[...], preferred_element_type=jnp.float32)
             + jnp.dot(agg, w2_ref[...], preferred_element_type=jnp.float32)
             + bn_ref[...])
        xn = x_ref[...] + _lrelu(y)
        o_ref[...] = xn
        pk_ref[...] = _pack_cols(xn)

    return pl.pallas_call(
        kern,
        grid=(grid,),
        in_specs=[pl.BlockSpec((blk, DIM), lambda i: (i, 0)),
                  pl.BlockSpec((2, blk, DIM), lambda i: (0, i, 0)),
                  pl.BlockSpec((2, blk, 16), lambda i: (0, i, 0)),
                  pl.BlockSpec((DIM, DIM), lambda i: (0, 0)),
                  pl.BlockSpec((DIM, DIM), lambda i: (0, 0)),
                  pl.BlockSpec((1, DIM), lambda i: (0, 0))],
        out_specs=(pl.BlockSpec((blk, DIM), lambda i: (i, 0)),
                   pl.BlockSpec((blk, PDIM), lambda i: (i, 0))),
        out_shape=(jax.ShapeDtypeStruct((N_NODES, DIM), jnp.float32),
                   jax.ShapeDtypeStruct((N_NODES, PDIM), jnp.float32)),
    )(x, parts, cnts, wn1, wn2, bn.reshape(1, DIM))


def _tc_pack(x):
    blk = 1000

    def kern(x_ref, o_ref):
        o_ref[...] = _pack_cols(x_ref[...])

    return pl.pallas_call(
        kern,
        grid=(N_NODES // blk,),
        in_specs=[pl.BlockSpec((blk, DIM), lambda i: (i, 0))],
        out_specs=pl.BlockSpec((blk, PDIM), lambda i: (i, 0)),
        out_shape=jax.ShapeDtypeStruct((N_NODES, PDIM), jnp.float32),
    )(x)


def _tc_head(os_, od, ea3, wa, wb, wc, b1, w2, b2):
    e = os_.shape[0]
    blk = 8000
    grid = e // blk

    def kern(os_ref, od_ref, ea_ref, wa_ref, wb_ref, wc_ref, b1_ref,
             w2_ref, b2_ref, o_ref):
        osl, osh = _unpack_cols(os_ref[...])
        odl, odh = _unpack_cols(od_ref[...])
        wa = wa_ref[...]
        wb = wb_ref[...]
        y = (jnp.dot(osl, wa[:PDIM], preferred_element_type=jnp.float32)
             + jnp.dot(osh, wa[PDIM:], preferred_element_type=jnp.float32)
             + jnp.dot(odl, wb[:PDIM], preferred_element_type=jnp.float32)
             + jnp.dot(odh, wb[PDIM:], preferred_element_type=jnp.float32)
             + jnp.dot(ea_ref[...], wc_ref[...], preferred_element_type=jnp.float32)
             + b1_ref[...])
        h = _lrelu(y)
        o_ref[...] = (jnp.dot(h, w2_ref[...], preferred_element_type=jnp.float32)
                      + b2_ref[...])

    row = lambda i: (i, 0)
    zero = lambda i: (0, 0)
    return pl.pallas_call(
        kern,
        grid=(grid,),
        in_specs=[pl.BlockSpec((blk, PDIM), row)] * 2 +
                 [pl.BlockSpec((blk, DIM), row)] +
                 [pl.BlockSpec((DIM, DIM), zero)] * 3 +
                 [pl.BlockSpec((1, DIM), zero),
                  pl.BlockSpec((DIM, 1), zero),
                  pl.BlockSpec((1, 1), zero)],
        out_specs=pl.BlockSpec((blk, 1), row),
        out_shape=jax.ShapeDtypeStruct((e, 1), jnp.float32),
    )(os_, od, ea3, wa, wb, wc, b1.reshape(1, DIM), w2, b2.reshape(1, 1))


def _sc_mesh():
    return plsc.VectorSubcoreMesh(core_axis_name="c", subcore_axis_name="s",
                                  num_cores=NC, num_subcores=NS)


_SC_PARAMS = pltpu.CompilerParams(use_tc_tiling_on_sc=False)


@functools.lru_cache(maxsize=None)
def _make_gather2(e):
    n_sg = e // (G * CHUNK)
    n_iter = -(-n_sg // NW)

    @functools.partial(
        pl.kernel,
        out_type=(jax.ShapeDtypeStruct((e, PDIM), jnp.float32),
                  jax.ShapeDtypeStruct((e, PDIM), jnp.float32)),
        mesh=_sc_mesh(),
        scratch_types=[pltpu.VMEM((G, CHUNK), jnp.int32),
                       pltpu.VMEM((G, CHUNK), jnp.int32),
                       pltpu.VMEM((G * CHUNK, PDIM), jnp.float32),
                       pltpu.VMEM((G * CHUNK, PDIM), jnp.float32),
                       pltpu.SemaphoreType.DMA,
                       pltpu.SemaphoreType.DMA,
                       pltpu.SemaphoreType.DMA],
        compiler_params=_SC_PARAMS,
    )
    def k(tab_hbm, src_hbm, dst_hbm, outs_hbm, outd_hbm,
          idxs_v, idxd_v, rows_v, rowd_v, sem_i, sem_g, sem_o):
        wid = lax.axis_index("s") * NC + lax.axis_index("c")

        def body(t, _):
            j = wid + t * NW

            @pl.when(j < n_sg)
            def _():
                row0 = j * G
                off = j * G * CHUNK
                ci = pltpu.async_copy(src_hbm.at[pl.ds(row0, G)], idxs_v, sem_i)
                cd = pltpu.async_copy(dst_hbm.at[pl.ds(row0, G)], idxd_v, sem_i)
                ci.wait()
                cd.wait()
                gs = [pltpu.async_copy(tab_hbm.at[idxs_v.at[g]],
                                       rows_v.at[pl.ds(g * CHUNK, CHUNK)],
                                       sem_g) for g in range(G)]
                gd = [pltpu.async_copy(tab_hbm.at[idxd_v.at[g]],
                                       rowd_v.at[pl.ds(g * CHUNK, CHUNK)],
                                       sem_g) for g in range(G)]
                for c in gs + gd:
                    c.wait()
                o1 = pltpu.async_copy(rows_v, outs_hbm.at[pl.ds(off, G * CHUNK)],
                                      sem_o)
                o2 = pltpu.async_copy(rowd_v, outd_hbm.at[pl.ds(off, G * CHUNK)],
                                      sem_o)
                o1.wait()
                o2.wait()

            return 0

        lax.fori_loop(0, n_iter, body, 0, unroll=False)

    return k


@functools.lru_cache(maxsize=None)
def _make_scatter_add(e):
    gs = 10
    n_sg = e // (gs * CHUNK)
    n_iter = -(-n_sg // NW)
    rows_per_tile = N_PAD // NS

    @functools.partial(
        pl.kernel,
        out_type=jax.ShapeDtypeStruct((NC, N_PAD, DIM), jnp.float32),
        mesh=_sc_mesh(),
        scratch_types=[pltpu.VMEM((gs, CHUNK), jnp.int32),
                       pltpu.VMEM((gs * CHUNK, DIM), jnp.float32),
                       pltpu.VMEM_SHARED((N_PAD, DIM), jnp.float32),
                       pltpu.SemaphoreType.DMA,
                       pltpu.SemaphoreType.DMA],
        compiler_params=_SC_PARAMS,
    )
    def k(vals_hbm, dst_hbm, zeros_hbm, out_hbm, idx_v, vals_v, acc_sh,
          sem_i, sem_s):
        cid = lax.axis_index("c")
        sid = lax.axis_index("s")
        wid = sid * NC + cid
        base = sid * rows_per_tile
        pltpu.sync_copy(zeros_hbm.at[pl.ds(base, rows_per_tile)],
                        acc_sh.at[pl.ds(base, rows_per_tile)])
        plsc.subcore_barrier()

        def body(t, _):
            j = wid + t * NW

            @pl.when(j < n_sg)
            def _():
                row0 = j * gs
                off = j * gs * CHUNK
                c1 = pltpu.async_copy(dst_hbm.at[pl.ds(row0, gs)], idx_v, sem_i)
                c2 = pltpu.async_copy(vals_hbm.at[pl.ds(off, gs * CHUNK)],
                                      vals_v, sem_i)
                c1.wait()
                c2.wait()
                cs = [pltpu.async_copy(vals_v.at[pl.ds(g * CHUNK, CHUNK)],
                                       acc_sh.at[idx_v.at[g]], sem_s, add=True)
                      for g in range(gs)]
                for c in cs:
                    c.wait()

            return 0

        lax.fori_loop(0, n_iter, body, 0, unroll=False)
        plsc.subcore_barrier()
        pltpu.sync_copy(acc_sh.at[pl.ds(base, rows_per_tile)],
                        out_hbm.at[cid, pl.ds(base, rows_per_tile)])

    return k


@functools.lru_cache(maxsize=None)
def _make_counts(e):
    gs = 10
    n_sg = e // (gs * CHUNK)
    n_iter = -(-n_sg // NW)
    rows_per_tile = 2 * N_PAD // NS

    @functools.partial(
        pl.kernel,
        out_type=jax.ShapeDtypeStruct((NC, 2 * N_PAD, 16), jnp.float32),
        mesh=_sc_mesh(),
        scratch_types=[pltpu.VMEM((gs, CHUNK), jnp.int32),
                       pltpu.VMEM((CHUNK, 16), jnp.float32),
                       pltpu.VMEM_SHARED((2 * N_PAD, 16), jnp.float32),
                       pltpu.SemaphoreType.DMA,
                       pltpu.SemaphoreType.DMA],
        compiler_params=_SC_PARAMS,
    )
    def k(dst_hbm, zeros_hbm, ones_hbm, out_hbm, idx_v, ones_v, acc_sh,
          sem_i, sem_s):
        cid = lax.axis_index("c")
        sid = lax.axis_index("s")
        wid = sid * NC + cid
        base = sid * rows_per_tile
        pltpu.sync_copy(zeros_hbm.at[pl.ds(base, rows_per_tile)],
                        acc_sh.at[pl.ds(base, rows_per_tile)])
        pltpu.sync_copy(ones_hbm, ones_v)
        plsc.subcore_barrier()

        def body(t, _):
            j = wid + t * NW

            @pl.when(j < n_sg)
            def _():
                row0 = j * gs
                c1 = pltpu.async_copy(dst_hbm.at[pl.ds(row0, gs)], idx_v, sem_i)
                c1.wait()
                cs = [pltpu.async_copy(ones_v, acc_sh.at[idx_v.at[g]], sem_s,
                                       add=True) for g in range(gs)]
                for c in cs:
                    c.wait()

            return 0

        lax.fori_loop(0, n_iter, body, 0, unroll=False)
        plsc.subcore_barrier()
        pltpu.sync_copy(acc_sh.at[pl.ds(base, rows_per_tile)],
                        out_hbm.at[cid, pl.ds(base, rows_per_tile)])

    return k


def _block(bp, x, xpk, src, dst, ea, cnts, zeros64):
    e = ea.shape[0]
    xs, xd = _make_gather2(e)(xpk, src, dst)
    we = bp['We']
    e_new, ea_out = _tc_edge(xs, xd, ea, we[0:DIM], we[DIM:2 * DIM],
                             we[2 * DIM:3 * DIM], bp['be'])
    parts = _make_scatter_add(e)(e_new, dst, zeros64)
    wn = bp['Wn']
    x_new, xpk_new = _tc_node(x, parts, cnts, wn[0:DIM], wn[DIM:2 * DIM],
                              bp['bn'])
    return x_new, xpk_new, ea_out


def kernel(x, edge_index, edge_attr, edge_index3, edge_attr3, edge_attr4,
           params):
    p = params
    n3 = edge_attr3.shape[0]

    out = _mlp(p['lin_node'], x)
    ea = _mlp(p['edge1'], edge_attr)
    temp = _mlp(p['edge2'], jnp.concatenate([edge_attr3, edge_attr4], axis=1))
    ea3 = jnp.concatenate([temp, temp], axis=0)

    src1 = edge_index[0].astype(jnp.int32).reshape(-1, CHUNK)
    dst1 = edge_index[1].astype(jnp.int32).reshape(-1, CHUNK)
    s3 = edge_index3[0].astype(jnp.int32)
    d3 = edge_index3[1].astype(jnp.int32)
    src2 = jnp.concatenate([s3, d3]).reshape(-1, CHUNK)
    dst2 = jnp.concatenate([d3, s3]).reshape(-1, CHUNK)
    s3 = s3.reshape(-1, CHUNK)
    d3 = d3.reshape(-1, CHUNK)

    zeros64 = jnp.zeros((N_PAD, DIM), jnp.float32)
    zeros16 = jnp.zeros((2 * N_PAD, 16), jnp.float32)
    ones16 = jnp.ones((CHUNK, 16), jnp.float32)

    e1 = edge_attr.shape[0]
    e2 = 2 * n3
    dst_all = jnp.concatenate([dst1.reshape(-1),
                               dst2.reshape(-1) + N_PAD]).reshape(-1, CHUNK)
    cnt_all = _make_counts(e1 + e2)(dst_all, zeros16, ones16)
    cnt1 = cnt_all[:, :N_PAD]
    cnt2 = cnt_all[:, N_PAD:]

    outpk = _tc_pack(out)
    for bp in p['conv1']:
        out, outpk, ea = _block(bp, out, outpk, src1, dst1, ea, cnt1, zeros64)
    for bp in p['conv2']:
        out, outpk, ea3 = _block(bp, out, outpk, src2, dst2, ea3, cnt2,
                                 zeros64)

    ea3 = ea3[:n3]
    os_, od = _make_gather2(n3)(outpk, s3, d3)
    w1 = p['head']['W1']
    yhat = _tc_head(os_, od, ea3, w1[0:DIM], w1[DIM:2 * DIM],
                    w1[2 * DIM:3 * DIM], p['head']['b1'],
                    p['head']['W2'], p['head']['b2'])
    return yhat[:, 0]


# speedup vs baseline: 1.0012x; 1.0012x over previous
import functools

import jax
import jax.numpy as jnp
from jax import lax
from jax.experimental import pallas as pl
from jax.experimental.pallas import tpu as pltpu
from jax.experimental.pallas import tpu_sc as plsc

N_NODES = 10000
N_PAD = 10240
DIM = 64
CHUNK = 128
G = 5
NC = 2
NS = 16
NW = NC * NS


PDIM = DIM // 2


def _lrelu(v):
    return jnp.where(v >= 0, v, 0.01 * v)


def _pack_cols(x):
    rnd = jnp.uint32(0x8000)
    lo = (lax.bitcast_convert_type(x[:, :PDIM], jnp.uint32) + rnd) >> 16
    hi = ((lax.bitcast_convert_type(x[:, PDIM:], jnp.uint32) + rnd)
          & jnp.uint32(0xFFFF0000))
    return lax.bitcast_convert_type(lo | hi, jnp.float32)


def _unpack_cols(pk):
    u = lax.bitcast_convert_type(pk, jnp.uint32)
    lo = lax.bitcast_convert_type(u << 16, jnp.float32)
    hi = lax.bitcast_convert_type(u & jnp.uint32(0xFFFF0000), jnp.float32)
    return lo, hi


def _tc_stats(x):
    r, d = x.shape
    blk = 8000 if r % 8000 == 0 else (2000 if r % 2000 == 0 else 1000)
    grid = r // blk

    def kern(x_ref, s_ref, q_ref):
        @pl.when(pl.program_id(0) == 0)
        def _():
            s_ref[...] = jnp.zeros_like(s_ref)
            q_ref[...] = jnp.zeros_like(q_ref)

        xb = x_ref[...]
        s = jnp.sum(xb, axis=0, keepdims=True)
        q = jnp.sum(xb * xb, axis=0, keepdims=True)
        s_ref[...] += jnp.broadcast_to(s, (8, d))
        q_ref[...] += jnp.broadcast_to(q, (8, d))

    return pl.pallas_call(
        kern,
        grid=(grid,),
        in_specs=[pl.BlockSpec((blk, d), lambda i: (i, 0))],
        out_specs=(pl.BlockSpec((8, d), lambda i: (0, 0)),
                   pl.BlockSpec((8, d), lambda i: (0, 0))),
        out_shape=(jax.ShapeDtypeStruct((8, d), jnp.float32),
                   jax.ShapeDtypeStruct((8, d), jnp.float32)),
    )(x)


def _tc_bn_mm(x, s, q, g, b, w, c):
    r, din = x.shape
    dout = w.shape[1]
    blk = 8000 if r % 8000 == 0 else (2000 if r % 2000 == 0 else 1000)
    grid = r // blk
    n = float(r)

    def kern(x_ref, s_ref, q_ref, g_ref, b_ref, w_ref, c_ref,
             o_ref, so_ref, qo_ref):
        @pl.when(pl.program_id(0) == 0)
        def _():
            so_ref[...] = jnp.zeros_like(so_ref)
            qo_ref[...] = jnp.zeros_like(qo_ref)

        mu = s_ref[...][0:1, :] / n
        var = q_ref[...][0:1, :] / n - mu * mu
        inv = lax.rsqrt(var + 1e-5)
        h = (x_ref[...] - mu) * (inv * g_ref[...]) + b_ref[...]
        y = jnp.dot(h, w_ref[...], preferred_element_type=jnp.float32)
        o = _lrelu(y + c_ref[...])
        o_ref[...] = o
        so_ref[...] += jnp.broadcast_to(jnp.sum(o, axis=0, keepdims=True),
                                        (8, dout))
        qo_ref[...] += jnp.broadcast_to(jnp.sum(o * o, axis=0, keepdims=True),
                                        (8, dout))

    full = lambda shape: pl.BlockSpec(shape, lambda i: tuple(0 for _ in shape))
    return pl.pallas_call(
        kern,
        grid=(grid,),
        in_specs=[pl.BlockSpec((blk, din), lambda i: (i, 0)),
                  full((8, din)), full((8, din)),
                  full((1, din)), full((1, din)),
                  full((din, dout)), full((1, dout))],
        out_specs=(pl.BlockSpec((blk, dout), lambda i: (i, 0)),
                   full((8, dout)), full((8, dout))),
        out_shape=(jax.ShapeDtypeStruct((r, dout), jnp.float32),
                   jax.ShapeDtypeStruct((8, dout), jnp.float32),
                   jax.ShapeDtypeStruct((8, dout), jnp.float32)),
    )(x, s, q, g.reshape(1, din), b.reshape(1, din), w, c.reshape(1, dout))


def _mlp(p, x):
    s, q = _tc_stats(x)
    h, s2, q2 = _tc_bn_mm(x, s, q, p['g1'], p['b1'], p['W1'], p['c1'])
    return _tc_bn_mm(h, s2, q2, p['g2'], p['b2'], p['W2'], p['c2'])[0]


def _tc_edge(xs, xd, ea, wa, wb, wc, be):
    e = xs.shape[0]
    blk = 8000
    grid = e // blk

    def kern(xs_ref, xd_ref, ea_ref, wa_ref, wb_ref, wc_ref, be_ref,
             en_ref, eo_ref):
        xsl, xsh = _unpack_cols(xs_ref[...])
        xdl, xdh = _unpack_cols(xd_ref[...])
        wa = wa_ref[...]
        wb = wb_ref[...]
        y = (jnp.dot(xsl, wa[:PDIM], preferred_element_type=jnp.float32)
             + jnp.dot(xsh, wa[PDIM:], preferred_element_type=jnp.float32)
             + jnp.dot(xdl, wb[:PDIM], preferred_element_type=jnp.float32)
             + jnp.dot(xdh, wb[PDIM:], preferred_element_type=jnp.float32)
             + jnp.dot(ea_ref[...], wc_ref[...], preferred_element_type=jnp.float32)
             + be_ref[...])
        en = _lrelu(y)
        en_ref[...] = en
        eo_ref[...] = ea_ref[...] + en

    row = lambda i: (i, 0)
    zero = lambda i: (0, 0)
    return pl.pallas_call(
        kern,
        grid=(grid,),
        in_specs=[pl.BlockSpec((blk, PDIM), row)] * 2 +
                 [pl.BlockSpec((blk, DIM), row)] +
                 [pl.BlockSpec((DIM, DIM), zero)] * 3 +
                 [pl.BlockSpec((1, DIM), zero)],
        out_specs=(pl.BlockSpec((blk, DIM), row), pl.BlockSpec((blk, DIM), row)),
        out_shape=(jax.ShapeDtypeStruct((e, DIM), jnp.float32),
                   jax.ShapeDtypeStruct((e, DIM), jnp.float32)),
    )(xs, xd, ea, wa, wb, wc, be.reshape(1, DIM))


def _tc_node(x, parts, cnts, wn1, wn2, bn):
    blk = 2000
    grid = N_NODES // blk

    def kern(x_ref, p_ref, c_ref, w1_ref, w2_ref, bn_ref, o_ref, pk_ref):
        psum = p_ref[0] + p_ref[1]
        cnt = (c_ref[0] + c_ref[1])[:, 0:1]
        agg = psum / jnp.maximum(cnt, 1.0)
        y = (jnp.dot(x_ref[...], w1_ref[...], preferred_element_type=jnp.float32)
             + jnp.dot(agg, w2_ref[...], preferred_element_type=jnp.float32)
             + bn_ref[...])
        xn = x_ref[...] + _lrelu(y)
        o_ref[...] = xn
        pk_ref[...] = _pack_cols(xn)

    return pl.pallas_call(
        kern,
        grid=(grid,),
        in_specs=[pl.BlockSpec((blk, DIM), lambda i: (i, 0)),
                  pl.BlockSpec((2, blk, DIM), lambda i: (0, i, 0)),
                  pl.BlockSpec((2, blk, 16), lambda i: (0, i, 0)),
                  pl.BlockSpec((DIM, DIM), lambda i: (0, 0)),
                  pl.BlockSpec((DIM, DIM), lambda i: (0, 0)),
                  pl.BlockSpec((1, DIM), lambda i: (0, 0))],
        out_specs=(pl.BlockSpec((blk, DIM), lambda i: (i, 0)),
                   pl.BlockSpec((blk, PDIM), lambda i: (i, 0))),
        out_shape=(jax.ShapeDtypeStruct((N_NODES, DIM), jnp.float32),
                   jax.ShapeDtypeStruct((N_NODES, PDIM), jnp.float32)),
    )(x, parts, cnts, wn1, wn2, bn.reshape(1, DIM))


def _tc_pack(x):
    blk = 1000

    def kern(x_ref, o_ref):
        o_ref[...] = _pack_cols(x_ref[...])

    return pl.pallas_call(
        kern,
        grid=(N_NODES // blk,),
        in_specs=[pl.BlockSpec((blk, DIM), lambda i: (i, 0))],
        out_specs=pl.BlockSpec((blk, PDIM), lambda i: (i, 0)),
        out_shape=jax.ShapeDtypeStruct((N_NODES, PDIM), jnp.float32),
    )(x)


def _tc_head(os_, od, ea3, wa, wb, wc, b1, w2, b2):
    e = os_.shape[0]
    blk = 8000
    grid = e // blk

    def kern(os_ref, od_ref, ea_ref, wa_ref, wb_ref, wc_ref, b1_ref,
             w2_ref, b2_ref, o_ref):
        osl, osh = _unpack_cols(os_ref[...])
        odl, odh = _unpack_cols(od_ref[...])
        wa = wa_ref[...]
        wb = wb_ref[...]
        y = (jnp.dot(osl, wa[:PDIM], preferred_element_type=jnp.float32)
             + jnp.dot(osh, wa[PDIM:], preferred_element_type=jnp.float32)
             + jnp.dot(odl, wb[:PDIM], preferred_element_type=jnp.float32)
             + jnp.dot(odh, wb[PDIM:], preferred_element_type=jnp.float32)
             + jnp.dot(ea_ref[...], wc_ref[...], preferred_element_type=jnp.float32)
             + b1_ref[...])
        h = _lrelu(y)
        o_ref[...] = (jnp.dot(h, w2_ref[...], preferred_element_type=jnp.float32)
                      + b2_ref[...])

    row = lambda i: (i, 0)
    zero = lambda i: (0, 0)
    return pl.pallas_call(
        kern,
        grid=(grid,),
        in_specs=[pl.BlockSpec((blk, PDIM), row)] * 2 +
                 [pl.BlockSpec((blk, DIM), row)] +
                 [pl.BlockSpec((DIM, DIM), zero)] * 3 +
                 [pl.BlockSpec((1, DIM), zero),
                  pl.BlockSpec((DIM, 1), zero),
                  pl.BlockSpec((1, 1), zero)],
        out_specs=pl.BlockSpec((blk, 1), row),
        out_shape=jax.ShapeDtypeStruct((e, 1), jnp.float32),
    )(os_, od, ea3, wa, wb, wc, b1.reshape(1, DIM), w2, b2.reshape(1, 1))


def _sc_mesh():
    return plsc.VectorSubcoreMesh(core_axis_name="c", subcore_axis_name="s",
                                  num_cores=NC, num_subcores=NS)


_SC_PARAMS = pltpu.CompilerParams(use_tc_tiling_on_sc=False)


@functools.lru_cache(maxsize=None)
def _make_gather2(e):
    n_sg = e // (G * CHUNK)
    n_iter = -(-n_sg // NW)

    @functools.partial(
        pl.kernel,
        out_type=(jax.ShapeDtypeStruct((e, PDIM), jnp.float32),
                  jax.ShapeDtypeStruct((e, PDIM), jnp.float32)),
        mesh=_sc_mesh(),
        scratch_types=[pltpu.VMEM((G, CHUNK), jnp.int32),
                       pltpu.VMEM((G, CHUNK), jnp.int32),
                       pltpu.VMEM((G * CHUNK, PDIM), jnp.float32),
                       pltpu.VMEM((G * CHUNK, PDIM), jnp.float32),
                       pltpu.SemaphoreType.DMA,
                       pltpu.SemaphoreType.DMA,
                       pltpu.SemaphoreType.DMA],
        compiler_params=_SC_PARAMS,
    )
    def k(tab_hbm, src_hbm, dst_hbm, outs_hbm, outd_hbm,
          idxs_v, idxd_v, rows_v, rowd_v, sem_i, sem_g, sem_o):
        wid = lax.axis_index("s") * NC + lax.axis_index("c")

        def body(t, _):
            j = wid + t * NW

            @pl.when(j < n_sg)
            def _():
                row0 = j * G
                off = j * G * CHUNK
                ci = pltpu.async_copy(src_hbm.at[pl.ds(row0, G)], idxs_v, sem_i)
                cd = pltpu.async_copy(dst_hbm.at[pl.ds(row0, G)], idxd_v, sem_i)
                ci.wait()
                cd.wait()
                gs = [pltpu.async_copy(tab_hbm.at[idxs_v.at[g]],
                                       rows_v.at[pl.ds(g * CHUNK, CHUNK)],
                                       sem_g) for g in range(G)]
                gd = [pltpu.async_copy(tab_hbm.at[idxd_v.at[g]],
                                       rowd_v.at[pl.ds(g * CHUNK, CHUNK)],
                                       sem_g) for g in range(G)]
                for c in gs + gd:
                    c.wait()
                o1 = pltpu.async_copy(rows_v, outs_hbm.at[pl.ds(off, G * CHUNK)],
                                      sem_o)
                o2 = pltpu.async_copy(rowd_v, outd_hbm.at[pl.ds(off, G * CHUNK)],
                                      sem_o)
                o1.wait()
                o2.wait()

            return 0

        lax.fori_loop(0, n_iter, body, 0, unroll=False)

    return k


@functools.lru_cache(maxsize=None)
def _make_scatter_add(e):
    gs = 10
    n_sg = e // (gs * CHUNK)
    n_iter = -(-n_sg // NW)
    rows_per_tile = N_PAD // NS

    @functools.partial(
        pl.kernel,
        out_type=jax.ShapeDtypeStruct((NC, N_PAD, DIM), jnp.float32),
        mesh=_sc_mesh(),
        scratch_types=[pltpu.VMEM((gs, CHUNK), jnp.int32),
                       pltpu.VMEM((gs * CHUNK, DIM), jnp.float32),
                       pltpu.VMEM_SHARED((N_PAD, DIM), jnp.float32),
                       pltpu.SemaphoreType.DMA,
                       pltpu.SemaphoreType.DMA],
        compiler_params=_SC_PARAMS,
    )
    def k(vals_hbm, dst_hbm, zeros_hbm, out_hbm, idx_v, vals_v, acc_sh,
          sem_i, sem_s):
        cid = lax.axis_index("c")
        sid = lax.axis_index("s")
        wid = sid * NC + cid
        base = sid * rows_per_tile
        pltpu.sync_copy(zeros_hbm.at[pl.ds(base, rows_per_tile)],
                        acc_sh.at[pl.ds(base, rows_per_tile)])
        plsc.subcore_barrier()

        def body(t, _):
            j = wid + t * NW

            @pl.when(j < n_sg)
            def _():
                row0 = j * gs
                off = j * gs * CHUNK
                c1 = pltpu.async_copy(dst_hbm.at[pl.ds(row0, gs)], idx_v, sem_i)
                c2 = pltpu.async_copy(vals_hbm.at[pl.ds(off, gs * CHUNK)],
                                      vals_v, sem_i)
                c1.wait()
                c2.wait()
                cs = [pltpu.async_copy(vals_v.at[pl.ds(g * CHUNK, CHUNK)],
                                       acc_sh.at[idx_v.at[g]], sem_s, add=True)
                      for g in range(gs)]
                for c in cs:
                    c.wait()

            return 0

        lax.fori_loop(0, n_iter, body, 0, unroll=False)
        plsc.subcore_barrier()
        pltpu.sync_copy(acc_sh.at[pl.ds(base, rows_per_tile)],
                        out_hbm.at[cid, pl.ds(base, rows_per_tile)])

    return k


@functools.lru_cache(maxsize=None)
def _make_counts(e):
    gs = 10
    n_sg = e // (gs * CHUNK)
    n_iter = -(-n_sg // NW)
    rows_per_tile = 2 * N_PAD // NS

    @functools.partial(
        pl.kernel,
        out_type=jax.ShapeDtypeStruct((NC, 2 * N_PAD, 16), jnp.float32),
        mesh=_sc_mesh(),
        scratch_types=[pltpu.VMEM((gs, CHUNK), jnp.int32),
                       pltpu.VMEM((CHUNK, 16), jnp.float32),
                       pltpu.VMEM_SHARED((2 * N_PAD, 16), jnp.float32),
                       pltpu.SemaphoreType.DMA,
                       pltpu.SemaphoreType.DMA],
        compiler_params=_SC_PARAMS,
    )
    def k(dst_hbm, zeros_hbm, ones_hbm, out_hbm, idx_v, ones_v, acc_sh,
          sem_i, sem_s):
        cid = lax.axis_index("c")
        sid = lax.axis_index("s")
        wid = sid * NC + cid
        base = sid * rows_per_tile
        pltpu.sync_copy(zeros_hbm.at[pl.ds(base, rows_per_tile)],
                        acc_sh.at[pl.ds(base, rows_per_tile)])
        pltpu.sync_copy(ones_hbm, ones_v)
        plsc.subcore_barrier()

        def body(t, _):
            j = wid + t * NW

            @pl.when(j < n_sg)
            def _():
                row0 = j * gs
                c1 = pltpu.async_copy(dst_hbm.at[pl.ds(row0, gs)], idx_v, sem_i)
                c1.wait()
                cs = [pltpu.async_copy(ones_v, acc_sh.at[idx_v.at[g]], sem_s,
                                       add=True) for g in range(gs)]
                for c in cs:
                    c.wait()

            return 0

        lax.fori_loop(0, n_iter, body, 0, unroll=False)
        plsc.subcore_barrier()
        pltpu.sync_copy(acc_sh.at[pl.ds(base, rows_per_tile)],
                        out_hbm.at[cid, pl.ds(base, rows_per_tile)])

    return k


def _block(bp, x, xpk, src, dst, ea, cnts, zeros64):
    e = ea.shape[0]
    xs, xd = _make_gather2(e)(xpk, src, dst)
    we = bp['We']
    e_new, ea_out = _tc_edge(xs, xd, ea, we[0:DIM], we[DIM:2 * DIM],
                             we[2 * DIM:3 * DIM], bp['be'])
    parts = _make_scatter_add(e)(e_new, dst, zeros64)
    wn = bp['Wn']
    x_new, xpk_new = _tc_node(x, parts, cnts, wn[0:DIM], wn[DIM:2 * DIM],
                              bp['bn'])
    return x_new, xpk_new, ea_out


def kernel(x, edge_index, edge_attr, edge_index3, edge_attr3, edge_attr4,
           params):
    p = params
    n3 = edge_attr3.shape[0]

    out = _mlp(p['lin_node'], x)
    ea = _mlp(p['edge1'], edge_attr)
    temp = _mlp(p['edge2'], jnp.concatenate([edge_attr3, edge_attr4], axis=1))
    ea3 = jnp.concatenate([temp, temp], axis=0)

    src1 = edge_index[0].astype(jnp.int32).reshape(-1, CHUNK)
    dst1 = edge_index[1].astype(jnp.int32).reshape(-1, CHUNK)
    s3 = edge_index3[0].astype(jnp.int32)
    d3 = edge_index3[1].astype(jnp.int32)
    src2 = jnp.concatenate([s3, d3]).reshape(-1, CHUNK)
    dst2 = jnp.concatenate([d3, s3]).reshape(-1, CHUNK)
    s3 = s3.reshape(-1, CHUNK)
    d3 = d3.reshape(-1, CHUNK)

    zeros64 = jnp.zeros((N_PAD, DIM), jnp.float32)
    zeros16 = jnp.zeros((2 * N_PAD, 16), jnp.float32)
    ones16 = jnp.ones((CHUNK, 16), jnp.float32)

    e1 = edge_attr.shape[0]
    e2 = 2 * n3
    dst_all = jnp.concatenate([dst1.reshape(-1),
                               dst2.reshape(-1) + N_PAD]).reshape(-1, CHUNK)
    cnt_all = _make_counts(e1 + e2)(dst_all, zeros16, ones16)
    cnt1 = cnt_all[:, :N_PAD]
    cnt2 = cnt_all[:, N_PAD:]

    outpk = _tc_pack(out)
    for bp in p['conv1']:
        out, outpk, ea = _block(bp, out, outpk, src1, dst1, ea, cnt1, zeros64)
    for bp in p['conv2']:
        out, outpk, ea3 = _block(bp, out, outpk, src2, dst2, ea3, cnt2,
                                 zeros64)

    ea3 = ea3[:n3]
    os_, od = _make_gather2(n3)(outpk, s3, d3)
    w1 = p['head']['W1']
    yhat = _tc_head(os_, od, ea3, w1[0:DIM], w1[DIM:2 * DIM],
                    w1[2 * DIM:3 * DIM], p['head']['b1'],
                    p['head']['W2'], p['head']['b2'])
    return yhat[:, 0]
